# Initial kernel scaffold; baseline (speedup 1.0000x reference)
import jax
import jax.numpy as jnp
from jax.experimental import pallas as pl


def kernel(x, edge_index, W0, b0, W1, b1, W2, b2, g0, beta0, g1, beta1):
    raise NotImplementedError("write your pallas kernel here")


# speedup vs baseline: 12.6518x; 12.6518x over previous
import functools

import jax
import jax.numpy as jnp
from jax import lax
from jax.experimental import pallas as pl
from jax.experimental.pallas import tpu as pltpu
from jax.experimental.pallas import tpu_sc as plsc

N = 10000
E = 320000
D = 128
EPS = 1e-5

NC = 2
NS = 16
NW = NC * NS
EPW = E // NW
CHUNK = 128
NFULL = EPW // CHUNK
TAIL = EPW - NFULL * CHUNK
STRIPE = 1000
NSTRIPE_TILES = N // STRIPE
SUBSTRIPE = 200

_mesh = plsc.VectorSubcoreMesh(
    core_axis_name="c", subcore_axis_name="s", num_cores=NC, num_subcores=NS
)


@functools.partial(
    pl.kernel,
    out_type=jax.ShapeDtypeStruct((NC * N,), jnp.float32),
    mesh=_mesh,
    scratch_types=[
        pltpu.VMEM((CHUNK,), jnp.int32),
        pltpu.VMEM((CHUNK,), jnp.float32),
        pltpu.VMEM((TAIL,), jnp.int32),
        pltpu.VMEM((TAIL,), jnp.float32),
        pltpu.VMEM((STRIPE,), jnp.float32),
        pltpu.VMEM_SHARED((N,), jnp.float32),
    ],
)
def _deg_kernel(dst_hbm, ones_hbm, zeros_hbm, out_hbm,
                idx_v, ones_v, idxt_v, onest_v, stage_v, acc_s):
    cid = lax.axis_index("c")
    sid = lax.axis_index("s")
    wid = cid * NS + sid

    @pl.when(sid < NSTRIPE_TILES)
    def _():
        rs = pl.ds(sid * STRIPE, STRIPE)
        pltpu.sync_copy(zeros_hbm.at[rs], stage_v)
        pltpu.sync_copy(stage_v, acc_s.at[rs])

    pltpu.sync_copy(ones_hbm.at[pl.ds(0, CHUNK)], ones_v)
    pltpu.sync_copy(ones_hbm.at[pl.ds(0, TAIL)], onest_v)
    plsc.subcore_barrier()

    base = wid * EPW

    def body(j, carry):
        pltpu.sync_copy(dst_hbm.at[pl.ds(base + j * CHUNK, CHUNK)], idx_v)
        pltpu.sync_copy(ones_v, acc_s.at[idx_v], add=True)
        return carry

    lax.fori_loop(0, NFULL, body, 0)
    pltpu.sync_copy(dst_hbm.at[pl.ds(base + NFULL * CHUNK, TAIL)], idxt_v)
    pltpu.sync_copy(onest_v, acc_s.at[idxt_v], add=True)
    plsc.subcore_barrier()

    @pl.when(sid < NSTRIPE_TILES)
    def _():
        rs = pl.ds(sid * STRIPE, STRIPE)
        pltpu.sync_copy(acc_s.at[rs], stage_v)
        pltpu.sync_copy(stage_v,
                        out_hbm.at[pl.ds(cid * N + sid * STRIPE, STRIPE)])


@functools.partial(
    pl.kernel,
    out_type=jax.ShapeDtypeStruct((NC, N, D), jnp.float32),
    mesh=_mesh,
    scratch_types=[
        pltpu.VMEM((CHUNK,), jnp.int32),
        pltpu.VMEM((CHUNK,), jnp.int32),
        pltpu.VMEM((CHUNK, D), jnp.float32),
        pltpu.VMEM((TAIL,), jnp.int32),
        pltpu.VMEM((TAIL,), jnp.int32),
        pltpu.VMEM((TAIL, D), jnp.float32),
        pltpu.VMEM((SUBSTRIPE, D), jnp.float32),
        pltpu.VMEM_SHARED((N, D), jnp.float32),
        pltpu.SemaphoreType.DMA,
    ],
)
def _agg_kernel(y_hbm, src_hbm, dst_hbm, zeros_hbm, out_hbm,
                si_v, di_v, rows_v, sit_v, dit_v, rowst_v, stage_v, acc_s,
                sem):
    cid = lax.axis_index("c")
    sid = lax.axis_index("s")
    wid = cid * NS + sid

    @pl.when(sid < NSTRIPE_TILES)
    def _():
        for j in range(STRIPE // SUBSTRIPE):
            rs = pl.ds(sid * STRIPE + j * SUBSTRIPE, SUBSTRIPE)

            @pl.when(cid == 0)
            def _():
                pltpu.sync_copy(y_hbm.at[rs], stage_v)

            @pl.when(cid == 1)
            def _():
                pltpu.sync_copy(zeros_hbm.at[rs], stage_v)

            pltpu.sync_copy(stage_v, acc_s.at[rs])

    plsc.subcore_barrier()

    base = wid * EPW

    def body(j, carry):
        off = base + j * CHUNK
        pltpu.sync_copy(src_hbm.at[pl.ds(off, CHUNK)], si_v)
        pltpu.sync_copy(dst_hbm.at[pl.ds(off, CHUNK)], di_v)
        pltpu.async_copy(y_hbm.at[si_v], rows_v, sem).wait()
        pltpu.sync_copy(rows_v, acc_s.at[di_v], add=True)
        return carry

    lax.fori_loop(0, NFULL, body, 0)

    offt = base + NFULL * CHUNK
    pltpu.sync_copy(src_hbm.at[pl.ds(offt, TAIL)], sit_v)
    pltpu.sync_copy(dst_hbm.at[pl.ds(offt, TAIL)], dit_v)
    pltpu.async_copy(y_hbm.at[sit_v], rowst_v, sem).wait()
    pltpu.sync_copy(rowst_v, acc_s.at[dit_v], add=True)
    plsc.subcore_barrier()

    @pl.when(sid < NSTRIPE_TILES)
    def _():
        for j in range(STRIPE // SUBSTRIPE):
            rs = pl.ds(sid * STRIPE + j * SUBSTRIPE, SUBSTRIPE)
            pltpu.sync_copy(acc_s.at[rs], stage_v)
            pltpu.sync_copy(stage_v, out_hbm.at[cid, rs])


def _mm_body(x_ref, w_ref, o_ref):
    o_ref[...] = lax.dot_general(
        x_ref[...], w_ref[...], (((1,), (0,)), ((), ())),
        precision=lax.Precision.HIGHEST, preferred_element_type=jnp.float32)


_mm = pl.pallas_call(
    _mm_body, out_shape=jax.ShapeDtypeStruct((N, D), jnp.float32))


def _scale_body(dp_ref, z_ref, dinv_ref, y_ref):
    deg = dp_ref[0] + dp_ref[1] + 1.0
    dinv = lax.rsqrt(jnp.maximum(deg, 1.0))
    dinv_ref[...] = dinv
    y_ref[...] = z_ref[...] * dinv


_scale = pl.pallas_call(
    _scale_body,
    out_shape=[
        jax.ShapeDtypeStruct((N, 1), jnp.float32),
        jax.ShapeDtypeStruct((N, D), jnp.float32),
    ])


def _layer_body(p_ref, dinv_ref, b_ref, g_ref, bt_ref, w_ref, y_ref):
    dinv = dinv_ref[...]
    v = (p_ref[0] + p_ref[1]) * dinv + b_ref[...]
    mu = jnp.mean(v, axis=0, keepdims=True)
    vc = v - mu
    var = jnp.mean(vc * vc, axis=0, keepdims=True)
    h = vc * lax.rsqrt(var + EPS) * g_ref[...] + bt_ref[...]
    h = jnp.maximum(h, 0.0)
    z = lax.dot_general(
        h, w_ref[...], (((1,), (0,)), ((), ())),
        precision=lax.Precision.HIGHEST, preferred_element_type=jnp.float32)
    y_ref[...] = z * dinv


_layer = pl.pallas_call(
    _layer_body, out_shape=jax.ShapeDtypeStruct((N, D), jnp.float32))


def _final_body(p_ref, dinv_ref, b_ref, o_ref):
    o_ref[...] = (p_ref[0] + p_ref[1]) * dinv_ref[...] + b_ref[...]


_final = pl.pallas_call(
    _final_body, out_shape=jax.ShapeDtypeStruct((N, D), jnp.float32))


def kernel(x, edge_index, W0, b0, W1, b1, W2, b2, g0, beta0, g1, beta1):
    src = edge_index[0]
    dst = edge_index[1]
    ones_c = jnp.ones((CHUNK,), jnp.float32)
    zeros_n = jnp.zeros((N,), jnp.float32)
    zeros_nd = jnp.zeros((N, D), jnp.float32)

    dp = _deg_kernel(dst, ones_c, zeros_n)
    z0 = _mm(x, W0)
    dinv, y0 = _scale(dp.reshape(NC, N, 1), z0)

    p = _agg_kernel(y0, src, dst, zeros_nd)
    y1 = _layer(p, dinv, b0.reshape(1, D), g0.reshape(1, D),
                beta0.reshape(1, D), W1)
    p = _agg_kernel(y1, src, dst, zeros_nd)
    y2 = _layer(p, dinv, b1.reshape(1, D), g1.reshape(1, D),
                beta1.reshape(1, D), W2)
    p = _agg_kernel(y2, src, dst, zeros_nd)
    return _final(p, dinv, b2.reshape(1, D))


# speedup vs baseline: 25.8863x; 2.0460x over previous
import functools

import jax
import jax.numpy as jnp
from jax import lax
from jax.experimental import pallas as pl
from jax.experimental.pallas import tpu as pltpu
from jax.experimental.pallas import tpu_sc as plsc

N = 10000
E = 320000
D = 128
EPS = 1e-5

NC = 2
NS = 16
NW = NC * NS
CHUNK = 128
NCH = 80
EP = NW * NCH * CHUNK
NPAD = EP - E
PADROWS = 240
NP = N + PADROWS
STRIPE = NP // NS
HOP = 128
NBUF = 2
NIDX = 4

_mesh = plsc.VectorSubcoreMesh(
    core_axis_name="c", subcore_axis_name="s", num_cores=NC, num_subcores=NS
)


@functools.partial(
    pl.kernel,
    out_type=jax.ShapeDtypeStruct((NC * NP,), jnp.float32),
    mesh=_mesh,
    scratch_types=[
        pltpu.VMEM((NCH, CHUNK), jnp.int32),
        pltpu.VMEM((CHUNK,), jnp.float32),
        pltpu.VMEM((STRIPE,), jnp.float32),
        pltpu.VMEM_SHARED((NP,), jnp.float32),
    ],
)
def _deg_kernel(dst_hbm, ones_hbm, zeros_hbm, out_hbm,
                slab_v, ones_v, stage_v, acc_s):
    cid = lax.axis_index("c")
    sid = lax.axis_index("s")
    wid = cid * NS + sid
    rs = pl.ds(sid * STRIPE, STRIPE)

    pltpu.sync_copy(zeros_hbm.at[rs], stage_v)
    pltpu.sync_copy(stage_v, acc_s.at[rs])
    pltpu.sync_copy(dst_hbm.at[wid], slab_v)
    pltpu.sync_copy(ones_hbm.at[pl.ds(0, CHUNK)], ones_v)
    plsc.subcore_barrier()

    def body(j, carry):
        pltpu.sync_copy(ones_v, acc_s.at[slab_v.at[j]], add=True)
        return carry

    lax.fori_loop(0, NCH, body, 0)
    plsc.subcore_barrier()

    pltpu.sync_copy(acc_s.at[rs], stage_v)
    pltpu.sync_copy(stage_v, out_hbm.at[pl.ds(cid * NP + sid * STRIPE, STRIPE)])


@functools.partial(
    pl.kernel,
    out_type=jax.ShapeDtypeStruct((NC, NP, D), jnp.float32),
    mesh=_mesh,
    scratch_types=[
        [pltpu.VMEM((2, CHUNK), jnp.int32)] * NIDX,
        [pltpu.SemaphoreType.DMA] * NIDX,
        [pltpu.VMEM((CHUNK, D), jnp.float32)] * NBUF,
        [pltpu.SemaphoreType.DMA] * NBUF,
        pltpu.VMEM_SHARED((NP, D), jnp.float32),
    ],
)
def _agg_kernel(y_hbm, idx_hbm, zeros_hbm, out_hbm,
                ibuf, isems, rows, rsems, acc_s):
    cid = lax.axis_index("c")
    sid = lax.axis_index("s")
    wid = cid * NS + sid

    for j in range(STRIPE // HOP):
        rs = pl.ds(sid * STRIPE + j * HOP, HOP)

        @pl.when(cid == 0)
        def _():
            pltpu.sync_copy(y_hbm.at[rs], rows[0])

        @pl.when(cid == 1)
        def _():
            pltpu.sync_copy(zeros_hbm.at[rs], rows[0])

        pltpu.sync_copy(rows[0], acc_s.at[rs])

    plsc.subcore_barrier()

    def _fire_idx(ch, k):
        pltpu.async_copy(idx_hbm.at[wid, ch], ibuf[k], isems[k])

    def _wait_idx(k):
        pltpu.make_async_copy(idx_hbm.at[0, 0], ibuf[k], isems[k]).wait()

    def _fire_gather(k, b):
        pltpu.async_copy(y_hbm.at[ibuf[k].at[0]], rows[b], rsems[b])

    def _wait_gather(b):
        pltpu.make_async_copy(y_hbm.at[pl.ds(0, CHUNK)], rows[b],
                              rsems[b]).wait()

    for k in range(NIDX):
        _fire_idx(k, k)
    for b in range(NBUF):
        _wait_idx(b)
        _fire_gather(b, b)

    def body(g, carry):
        for p in range(NIDX):
            ch = NIDX * g + p
            b = p % NBUF
            _wait_gather(b)
            pltpu.sync_copy(rows[b], acc_s.at[ibuf[p].at[1]], add=True)

            @pl.when(ch + NIDX < NCH)
            def _():
                _fire_idx(ch + NIDX, p)

            @pl.when(ch + NBUF < NCH)
            def _():
                _wait_idx((p + NBUF) % NIDX)
                _fire_gather((p + NBUF) % NIDX, b)
        return carry

    lax.fori_loop(0, NCH // NIDX, body, 0)
    plsc.subcore_barrier()

    for j in range(STRIPE // HOP):
        rs = pl.ds(sid * STRIPE + j * HOP, HOP)
        pltpu.sync_copy(acc_s.at[rs], rows[0])
        pltpu.sync_copy(rows[0], out_hbm.at[cid, rs])


def _mm_body(x_ref, w_ref, o_ref):
    o_ref[...] = lax.dot_general(
        x_ref[...], w_ref[...], (((1,), (0,)), ((), ())),
        precision=lax.Precision.HIGHEST, preferred_element_type=jnp.float32)


_mm = pl.pallas_call(
    _mm_body, out_shape=jax.ShapeDtypeStruct((N, D), jnp.float32))


def _scale_body(dp_ref, z_ref, dinv_ref, y_ref):
    dp = dp_ref[...]
    deg = dp[0, :N] + dp[1, :N] + 1.0
    dinv = lax.rsqrt(jnp.maximum(deg, 1.0))
    dinv_ref[...] = dinv
    y_ref[0:N, :] = z_ref[...] * dinv
    y_ref[N:NP, :] = jnp.zeros((NP - N, D), jnp.float32)


_scale = pl.pallas_call(
    _scale_body,
    out_shape=[
        jax.ShapeDtypeStruct((N, 1), jnp.float32),
        jax.ShapeDtypeStruct((NP, D), jnp.float32),
    ])


def _layer_body(p_ref, dinv_ref, b_ref, g_ref, bt_ref, w_ref, y_ref):
    dinv = dinv_ref[...]
    s = p_ref[0] + p_ref[1]
    v = s[:N] * dinv + b_ref[...]
    mu = jnp.mean(v, axis=0, keepdims=True)
    vc = v - mu
    var = jnp.mean(vc * vc, axis=0, keepdims=True)
    h = vc * lax.rsqrt(var + EPS) * g_ref[...] + bt_ref[...]
    h = jnp.maximum(h, 0.0)
    z = lax.dot_general(
        h, w_ref[...], (((1,), (0,)), ((), ())),
        precision=lax.Precision.HIGHEST, preferred_element_type=jnp.float32)
    y_ref[0:N, :] = z * dinv
    y_ref[N:NP, :] = jnp.zeros((NP - N, D), jnp.float32)


_layer = pl.pallas_call(
    _layer_body, out_shape=jax.ShapeDtypeStruct((NP, D), jnp.float32))


def _final_body(p_ref, dinv_ref, b_ref, o_ref):
    s = p_ref[0] + p_ref[1]
    o_ref[...] = s[:N] * dinv_ref[...] + b_ref[...]


_final = pl.pallas_call(
    _final_body, out_shape=jax.ShapeDtypeStruct((N, D), jnp.float32))


def kernel(x, edge_index, W0, b0, W1, b1, W2, b2, g0, beta0, g1, beta1):
    src = edge_index[0]
    dst = edge_index[1]
    pad = jnp.arange(NPAD, dtype=jnp.int32)
    srcp = jnp.concatenate([src, (pad * 37) % N]).reshape(NW, NCH, CHUNK)
    dstp = jnp.concatenate([dst, N + pad % PADROWS]).reshape(NW, NCH, CHUNK)
    idxp = jnp.stack([srcp, dstp], axis=2)
    ones_c = jnp.ones((CHUNK,), jnp.float32)
    zeros_n = jnp.zeros((NP,), jnp.float32)
    zeros_nd = jnp.zeros((NP, D), jnp.float32)

    dp = _deg_kernel(dstp, ones_c, zeros_n)
    z0 = _mm(x, W0)
    dinv, y0 = _scale(dp.reshape(NC, NP, 1), z0)

    p = _agg_kernel(y0, idxp, zeros_nd)
    y1 = _layer(p, dinv, b0.reshape(1, D), g0.reshape(1, D),
                beta0.reshape(1, D), W1)
    p = _agg_kernel(y1, idxp, zeros_nd)
    y2 = _layer(p, dinv, b1.reshape(1, D), g1.reshape(1, D),
                beta1.reshape(1, D), W2)
    p = _agg_kernel(y2, idxp, zeros_nd)
    return _final(p, dinv, b2.reshape(1, D))


# speedup vs baseline: 26.6696x; 1.0303x over previous
import functools

import jax
import jax.numpy as jnp
from jax import lax
from jax.experimental import pallas as pl
from jax.experimental.pallas import tpu as pltpu
from jax.experimental.pallas import tpu_sc as plsc

N = 10000
E = 320000
D = 128
EPS = 1e-5

NC = 2
NS = 16
NW = NC * NS
CHUNK = 128
NCH = 80
EP = NW * NCH * CHUNK
NPAD = EP - E
PADROWS = 240
NP = N + PADROWS
STRIPE = NP // NS
HOP = 128
NBUF = 2
NIDX = 4

_mesh = plsc.VectorSubcoreMesh(
    core_axis_name="c", subcore_axis_name="s", num_cores=NC, num_subcores=NS
)


@functools.partial(
    pl.kernel,
    out_type=jax.ShapeDtypeStruct((NC * NP,), jnp.float32),
    mesh=_mesh,
    scratch_types=[
        pltpu.VMEM((NCH, CHUNK), jnp.int32),
        pltpu.VMEM((CHUNK,), jnp.float32),
        pltpu.VMEM((STRIPE,), jnp.float32),
        pltpu.VMEM_SHARED((NP,), jnp.float32),
    ],
)
def _deg_kernel(dst_hbm, ones_hbm, zeros_hbm, out_hbm,
                slab_v, ones_v, stage_v, acc_s):
    cid = lax.axis_index("c")
    sid = lax.axis_index("s")
    wid = cid * NS + sid
    rs = pl.ds(sid * STRIPE, STRIPE)

    pltpu.sync_copy(zeros_hbm.at[rs], stage_v)
    pltpu.sync_copy(stage_v, acc_s.at[rs])
    pltpu.sync_copy(dst_hbm.at[wid], slab_v)
    pltpu.sync_copy(ones_hbm.at[pl.ds(0, CHUNK)], ones_v)
    plsc.subcore_barrier()

    def body(j, carry):
        pltpu.sync_copy(ones_v, acc_s.at[slab_v.at[j]], add=True)
        return carry

    lax.fori_loop(0, NCH, body, 0)
    plsc.subcore_barrier()

    pltpu.sync_copy(acc_s.at[rs], stage_v)
    pltpu.sync_copy(stage_v, out_hbm.at[pl.ds(cid * NP + sid * STRIPE, STRIPE)])


@functools.partial(
    pl.kernel,
    out_type=jax.ShapeDtypeStruct((NC, NP, D), jnp.float32),
    mesh=_mesh,
    scratch_types=[
        [pltpu.VMEM((2, CHUNK), jnp.int32)] * NIDX,
        [pltpu.SemaphoreType.DMA] * NIDX,
        [pltpu.VMEM((CHUNK, D), jnp.float32)] * NBUF,
        [pltpu.SemaphoreType.DMA] * NBUF,
        pltpu.VMEM_SHARED((NP, D), jnp.float32),
    ],
)
def _agg_kernel(y_hbm, idx_hbm, zeros_hbm, out_hbm,
                ibuf, isems, rows, rsems, acc_s):
    cid = lax.axis_index("c")
    sid = lax.axis_index("s")
    wid = cid * NS + sid

    nhop = STRIPE // HOP

    def _hop_src(j):
        return pl.ds(sid * STRIPE + j * HOP, HOP)

    def _fire_hop(j, b):
        @pl.when(cid == 0)
        def _():
            pltpu.async_copy(y_hbm.at[_hop_src(j)], rows[b], rsems[b])

        @pl.when(cid == 1)
        def _():
            pltpu.async_copy(zeros_hbm.at[_hop_src(j)], rows[b], rsems[b])

    _fire_hop(0, 0)
    for j in range(nhop):
        b = j % NBUF
        if j + 1 < nhop:
            _fire_hop(j + 1, (j + 1) % NBUF)
        pltpu.make_async_copy(y_hbm.at[pl.ds(0, HOP)], rows[b],
                              rsems[b]).wait()
        pltpu.sync_copy(rows[b], acc_s.at[_hop_src(j)])

    plsc.subcore_barrier()

    def _fire_idx(ch, k):
        pltpu.async_copy(idx_hbm.at[wid, ch], ibuf[k], isems[k])

    def _wait_idx(k):
        pltpu.make_async_copy(idx_hbm.at[0, 0], ibuf[k], isems[k]).wait()

    def _fire_gather(k, b):
        pltpu.async_copy(y_hbm.at[ibuf[k].at[0]], rows[b], rsems[b])

    def _wait_gather(b):
        pltpu.make_async_copy(y_hbm.at[pl.ds(0, CHUNK)], rows[b],
                              rsems[b]).wait()

    for k in range(NIDX):
        _fire_idx(k, k)
    for b in range(NBUF):
        _wait_idx(b)
        _fire_gather(b, b)

    def body(g, carry):
        for p in range(NIDX):
            ch = NIDX * g + p
            b = p % NBUF
            _wait_gather(b)
            pltpu.sync_copy(rows[b], acc_s.at[ibuf[p].at[1]], add=True)

            @pl.when(ch + NIDX < NCH)
            def _():
                _fire_idx(ch + NIDX, p)

            @pl.when(ch + NBUF < NCH)
            def _():
                _wait_idx((p + NBUF) % NIDX)
                _fire_gather((p + NBUF) % NIDX, b)
        return carry

    lax.fori_loop(0, NCH // NIDX, body, 0)
    plsc.subcore_barrier()

    for j in range(nhop):
        b = j % NBUF
        rs = _hop_src(j)
        if j >= NBUF:
            pltpu.make_async_copy(y_hbm.at[pl.ds(0, HOP)], rows[b],
                                  rsems[b]).wait()
        pltpu.sync_copy(acc_s.at[rs], rows[b])
        pltpu.async_copy(rows[b], out_hbm.at[cid, rs], rsems[b])
    for j in range(NBUF):
        pltpu.make_async_copy(y_hbm.at[pl.ds(0, HOP)], rows[j],
                              rsems[j]).wait()


def _scale_body(dp_ref, x_ref, w_ref, dinv_ref, y_ref):
    dp = dp_ref[...]
    deg = dp[0, :N] + dp[1, :N] + 1.0
    dinv = lax.rsqrt(jnp.maximum(deg, 1.0))
    dinv_ref[...] = dinv
    z = lax.dot_general(
        x_ref[...], w_ref[...], (((1,), (0,)), ((), ())),
        precision=lax.Precision.HIGHEST, preferred_element_type=jnp.float32)
    y_ref[0:N, :] = z * dinv
    y_ref[N:NP, :] = jnp.zeros((NP - N, D), jnp.float32)


_scale = pl.pallas_call(
    _scale_body,
    out_shape=[
        jax.ShapeDtypeStruct((N, 1), jnp.float32),
        jax.ShapeDtypeStruct((NP, D), jnp.float32),
    ])


def _layer_body(p_ref, dinv_ref, b_ref, g_ref, bt_ref, w_ref, y_ref):
    dinv = dinv_ref[...]
    s = p_ref[0] + p_ref[1]
    v = s[:N] * dinv + b_ref[...]
    mu = jnp.mean(v, axis=0, keepdims=True)
    vc = v - mu
    var = jnp.mean(vc * vc, axis=0, keepdims=True)
    h = vc * lax.rsqrt(var + EPS) * g_ref[...] + bt_ref[...]
    h = jnp.maximum(h, 0.0)
    z = lax.dot_general(
        h, w_ref[...], (((1,), (0,)), ((), ())),
        precision=lax.Precision.HIGHEST, preferred_element_type=jnp.float32)
    y_ref[0:N, :] = z * dinv
    y_ref[N:NP, :] = jnp.zeros((NP - N, D), jnp.float32)


_layer = pl.pallas_call(
    _layer_body, out_shape=jax.ShapeDtypeStruct((NP, D), jnp.float32))


def _final_body(p_ref, dinv_ref, b_ref, o_ref):
    s = p_ref[0] + p_ref[1]
    o_ref[...] = s[:N] * dinv_ref[...] + b_ref[...]


_final = pl.pallas_call(
    _final_body, out_shape=jax.ShapeDtypeStruct((N, D), jnp.float32))


def kernel(x, edge_index, W0, b0, W1, b1, W2, b2, g0, beta0, g1, beta1):
    src = edge_index[0]
    dst = edge_index[1]
    pad = jnp.arange(NPAD, dtype=jnp.int32)
    srcp = jnp.concatenate([src, (pad * 37) % N]).reshape(NW, NCH, CHUNK)
    dstp = jnp.concatenate([dst, N + pad % PADROWS]).reshape(NW, NCH, CHUNK)
    idxp = jnp.stack([srcp, dstp], axis=2)
    ones_c = jnp.ones((CHUNK,), jnp.float32)
    zeros_n = jnp.zeros((NP,), jnp.float32)
    zeros_nd = jnp.zeros((NP, D), jnp.float32)

    dp = _deg_kernel(dstp, ones_c, zeros_n)
    dinv, y0 = _scale(dp.reshape(NC, NP, 1), x, W0)

    p = _agg_kernel(y0, idxp, zeros_nd)
    y1 = _layer(p, dinv, b0.reshape(1, D), g0.reshape(1, D),
                beta0.reshape(1, D), W1)
    p = _agg_kernel(y1, idxp, zeros_nd)
    y2 = _layer(p, dinv, b1.reshape(1, D), g1.reshape(1, D),
                beta1.reshape(1, D), W2)
    p = _agg_kernel(y2, idxp, zeros_nd)
    return _final(p, dinv, b2.reshape(1, D))


# speedup vs baseline: 27.5644x; 1.0336x over previous
import functools

import jax
import jax.numpy as jnp
from jax import lax
from jax.experimental import pallas as pl
from jax.experimental.pallas import tpu as pltpu
from jax.experimental.pallas import tpu_sc as plsc

N = 10000
E = 320000
D = 128
EPS = 1e-5

NC = 2
NS = 16
NW = NC * NS
CHUNK = 128
NCH = 80
EP = NW * NCH * CHUNK
EPW = NCH * CHUNK
NPAD = EP - E
PADROWS = 240
NP = N + PADROWS
STRIPE = NP // NS
HOP = 128
NBUF = 2
NIDX = 4

_mesh = plsc.VectorSubcoreMesh(
    core_axis_name="c", subcore_axis_name="s", num_cores=NC, num_subcores=NS
)


@functools.partial(
    pl.kernel,
    out_type=jax.ShapeDtypeStruct((NC * NP,), jnp.float32),
    mesh=_mesh,
    scratch_types=[
        pltpu.VMEM((NCH, CHUNK), jnp.int32),
        pltpu.VMEM((CHUNK,), jnp.float32),
        pltpu.VMEM((STRIPE,), jnp.float32),
        pltpu.VMEM_SHARED((NP,), jnp.float32),
    ],
)
def _deg_kernel(dst_hbm, ones_hbm, zeros_hbm, out_hbm,
                slab_v, ones_v, stage_v, acc_s):
    cid = lax.axis_index("c")
    sid = lax.axis_index("s")
    wid = cid * NS + sid
    rs = pl.ds(sid * STRIPE, STRIPE)

    pltpu.sync_copy(zeros_hbm.at[rs], stage_v)
    pltpu.sync_copy(stage_v, acc_s.at[rs])
    pltpu.sync_copy(dst_hbm.at[wid], slab_v)
    pltpu.sync_copy(ones_hbm.at[pl.ds(0, CHUNK)], ones_v)
    plsc.subcore_barrier()

    def body(j, carry):
        pltpu.sync_copy(ones_v, acc_s.at[slab_v.at[j]], add=True)
        return carry

    lax.fori_loop(0, NCH, body, 0)
    plsc.subcore_barrier()

    pltpu.sync_copy(acc_s.at[rs], stage_v)
    pltpu.sync_copy(stage_v, out_hbm.at[pl.ds(cid * NP + sid * STRIPE, STRIPE)])


@functools.partial(
    pl.kernel,
    out_type=jax.ShapeDtypeStruct((NC, NP, D), jnp.float32),
    mesh=_mesh,
    scratch_types=[
        [pltpu.VMEM((2, CHUNK), jnp.int32)] * NIDX,
        [pltpu.SemaphoreType.DMA] * NIDX,
        [pltpu.VMEM((CHUNK, D), jnp.float32)] * NBUF,
        [pltpu.SemaphoreType.DMA] * NBUF,
        pltpu.VMEM_SHARED((NP, D), jnp.float32),
    ],
)
def _agg_kernel(y_hbm, src_hbm, dst_hbm, zeros_hbm, out_hbm,
                ibuf, isems, rows, rsems, acc_s):
    cid = lax.axis_index("c")
    sid = lax.axis_index("s")
    wid = cid * NS + sid

    nhop = STRIPE // HOP

    def _hop_src(j):
        return pl.ds(sid * STRIPE + j * HOP, HOP)

    def _fire_hop(j, b):
        @pl.when(cid == 0)
        def _():
            pltpu.async_copy(y_hbm.at[_hop_src(j)], rows[b], rsems[b])

        @pl.when(cid == 1)
        def _():
            pltpu.async_copy(zeros_hbm.at[_hop_src(j)], rows[b], rsems[b])

    _fire_hop(0, 0)
    for j in range(nhop):
        b = j % NBUF
        if j + 1 < nhop:
            _fire_hop(j + 1, (j + 1) % NBUF)
        pltpu.make_async_copy(y_hbm.at[pl.ds(0, HOP)], rows[b],
                              rsems[b]).wait()
        pltpu.sync_copy(rows[b], acc_s.at[_hop_src(j)])

    plsc.subcore_barrier()

    def _fire_idx(ch, k):
        off = wid * EPW + ch * CHUNK
        pltpu.async_copy(src_hbm.at[pl.ds(off, CHUNK)], ibuf[k].at[0],
                         isems[k])
        pltpu.async_copy(dst_hbm.at[pl.ds(off, CHUNK)], ibuf[k].at[1],
                         isems[k])

    def _wait_idx(k):
        pltpu.make_async_copy(src_hbm.at[pl.ds(0, CHUNK)], ibuf[k].at[0],
                              isems[k]).wait()
        pltpu.make_async_copy(src_hbm.at[pl.ds(0, CHUNK)], ibuf[k].at[1],
                              isems[k]).wait()

    def _fire_gather(k, b):
        pltpu.async_copy(y_hbm.at[ibuf[k].at[0]], rows[b], rsems[b])

    def _wait_gather(b):
        pltpu.make_async_copy(y_hbm.at[pl.ds(0, CHUNK)], rows[b],
                              rsems[b]).wait()

    for k in range(NIDX):
        _fire_idx(k, k)
    for b in range(NBUF):
        _wait_idx(b)
        _fire_gather(b, b)

    def body(g, carry):
        for p in range(NIDX):
            ch = NIDX * g + p
            b = p % NBUF
            _wait_gather(b)
            pltpu.sync_copy(rows[b], acc_s.at[ibuf[p].at[1]], add=True)

            @pl.when(ch + NIDX < NCH)
            def _():
                _fire_idx(ch + NIDX, p)

            @pl.when(ch + NBUF < NCH)
            def _():
                _wait_idx((p + NBUF) % NIDX)
                _fire_gather((p + NBUF) % NIDX, b)
        return carry

    lax.fori_loop(0, NCH // NIDX, body, 0)
    plsc.subcore_barrier()

    for j in range(nhop):
        b = j % NBUF
        rs = _hop_src(j)
        if j >= NBUF:
            pltpu.make_async_copy(y_hbm.at[pl.ds(0, HOP)], rows[b],
                                  rsems[b]).wait()
        pltpu.sync_copy(acc_s.at[rs], rows[b])
        pltpu.async_copy(rows[b], out_hbm.at[cid, rs], rsems[b])
    for j in range(NBUF):
        pltpu.make_async_copy(y_hbm.at[pl.ds(0, HOP)], rows[j],
                              rsems[j]).wait()


def _scale_body(dp_ref, x_ref, w_ref, dinv_ref, y_ref):
    dp = dp_ref[...]
    deg = dp[0:N] + dp[NP:NP + N] + 1.0
    dinv = lax.rsqrt(jnp.maximum(deg, 1.0))[:, None]
    dinv_ref[...] = dinv
    z = lax.dot_general(
        x_ref[...], w_ref[...], (((1,), (0,)), ((), ())),
        precision=lax.Precision.HIGHEST, preferred_element_type=jnp.float32)
    y_ref[0:N, :] = z * dinv
    y_ref[N:NP, :] = jnp.zeros((NP - N, D), jnp.float32)


_scale = pl.pallas_call(
    _scale_body,
    out_shape=[
        jax.ShapeDtypeStruct((N, 1), jnp.float32),
        jax.ShapeDtypeStruct((NP, D), jnp.float32),
    ])


def _layer_body(p_ref, dinv_ref, b_ref, g_ref, bt_ref, w_ref, y_ref):
    dinv = dinv_ref[...]
    s = p_ref[0] + p_ref[1]
    v = s[:N] * dinv + b_ref[...]
    mu = jnp.mean(v, axis=0, keepdims=True)
    vc = v - mu
    var = jnp.mean(vc * vc, axis=0, keepdims=True)
    h = vc * lax.rsqrt(var + EPS) * g_ref[...] + bt_ref[...]
    h = jnp.maximum(h, 0.0)
    z = lax.dot_general(
        h, w_ref[...], (((1,), (0,)), ((), ())),
        precision=lax.Precision.HIGHEST, preferred_element_type=jnp.float32)
    y_ref[0:N, :] = z * dinv
    y_ref[N:NP, :] = jnp.zeros((NP - N, D), jnp.float32)


_layer = pl.pallas_call(
    _layer_body, out_shape=jax.ShapeDtypeStruct((NP, D), jnp.float32))


def _final_body(p_ref, dinv_ref, b_ref, o_ref):
    s = p_ref[0] + p_ref[1]
    o_ref[...] = s[:N] * dinv_ref[...] + b_ref[...]


_final = pl.pallas_call(
    _final_body, out_shape=jax.ShapeDtypeStruct((N, D), jnp.float32))


def kernel(x, edge_index, W0, b0, W1, b1, W2, b2, g0, beta0, g1, beta1):
    src = edge_index[0]
    dst = edge_index[1]
    pad = jnp.arange(NPAD, dtype=jnp.int32)
    srcp = jnp.concatenate([src, (pad * 37) % N])
    dstp = jnp.concatenate([dst, N + pad % PADROWS])
    dstp3 = dstp.reshape(NW, NCH, CHUNK)
    ones_c = jnp.ones((CHUNK,), jnp.float32)
    zeros_n = jnp.zeros((NP,), jnp.float32)
    zeros_nd = jnp.zeros((NP, D), jnp.float32)

    dp = _deg_kernel(dstp3, ones_c, zeros_n)
    dinv, y0 = _scale(dp, x, W0)

    p = _agg_kernel(y0, srcp, dstp, zeros_nd)
    y1 = _layer(p, dinv, b0.reshape(1, D), g0.reshape(1, D),
                beta0.reshape(1, D), W1)
    p = _agg_kernel(y1, srcp, dstp, zeros_nd)
    y2 = _layer(p, dinv, b1.reshape(1, D), g1.reshape(1, D),
                beta1.reshape(1, D), W2)
    p = _agg_kernel(y2, srcp, dstp, zeros_nd)
    return _final(p, dinv, b2.reshape(1, D))
